# baseline (speedup 1.0000x reference)
import functools

import jax
import jax.numpy as jnp
from jax import lax
from jax.experimental import pallas as pl
from jax.experimental.pallas import tpu as pltpu
from jax.experimental.pallas import tpu_sc as plsc

N_EXPERTS = 8
TOP_K = 2
NOISE_SCALE = 50.0

_noise_cache = {}


def _get_noise_t(tokens: int):
    if tokens not in _noise_cache:
        noise = jax.random.normal(
            jax.random.key(1), (tokens, N_EXPERTS), dtype=jnp.float32
        ) * NOISE_SCALE
        _noise_cache[tokens] = noise.T.copy()
    return _noise_cache[tokens]


def _make_scores_body(q, per_w, step):
    wpb = step // per_w

    def _scores_body(*refs):
        x_refs = refs[:q]
        w_ref, n_ref, o_ref = refs[q], refs[q + 1], refs[q + 2]
        w = w_ref[...]
        parts = [
            lax.dot_general(
                w, x_ref[0],
                (((1,), (1,)), ((), ())),
                preferred_element_type=jnp.float32,
            )
            for x_ref in x_refs
        ]
        logits = (parts[0] if q == 1 else jnp.concatenate(parts, axis=1)) + n_ref[...]
        m = jnp.max(logits, axis=0, keepdims=True)
        e = jnp.exp(logits - m)
        s = jnp.sum(e, axis=0, keepdims=True)
        scores = e / s
        for wk in range(wpb):
            o_ref[wk] = scores[:, wk * per_w:(wk + 1) * per_w]
    return _scores_body


@functools.partial(jax.jit, static_argnames=("blk", "q", "per_w"))
def _tc_scores_t(hidden_states, weight, noise_t, per_w, blk=1024, q=4):
    bsz, seq, h = hidden_states.shape
    step = q * blk
    sb = seq // step
    x_specs = [
        pl.BlockSpec((1, blk, h), lambda i, j, p=p: (i, j * q + p, 0))
        for p in range(q)
    ]
    return pl.pallas_call(
        _make_scores_body(q, per_w, step),
        grid=(bsz, sb),
        in_specs=x_specs + [
            pl.BlockSpec((N_EXPERTS, h), lambda i, j: (0, 0)),
            pl.BlockSpec((N_EXPERTS, step), lambda i, j: (0, i * sb + j)),
        ],
        out_specs=pl.BlockSpec(
            (step // per_w, N_EXPERTS, per_w), lambda i, j: (i * sb + j, 0, 0)),
        out_shape=jax.ShapeDtypeStruct(
            (bsz * seq // per_w, N_EXPERTS, per_w), jnp.float32),
        compiler_params=pltpu.CompilerParams(
            dimension_semantics=("parallel", "parallel")),
    )(*([hidden_states] * q), weight, noise_t)


def _route_body(t, per_w, scores_hbm, i1_hbm, i2_hbm, w1_hbm, w2_hbm,
                sc_v, i1_v, i2_v, w1_v, w2_v):
    wid = lax.axis_index("s") * 2 + lax.axis_index("c")
    base = wid * per_w
    pltpu.sync_copy(
        scores_hbm.at[pl.ds(base * N_EXPERTS, per_w * N_EXPERTS)], sc_v)

    neg_one = jnp.full((16,), -1.0, jnp.float32)

    def group(g, _):
        o = g * 16
        vals = [sc_v[pl.ds(e * per_w + o, 16)] for e in range(N_EXPERTS)]
        sl = pl.ds(o, 16)
        m1 = vals[0]
        for v in vals[1:]:
            m1 = jnp.maximum(m1, v)
        a1 = jnp.full((16,), N_EXPERTS - 1, jnp.int32)
        for e in range(N_EXPERTS - 2, -1, -1):
            a1 = jnp.where(vals[e] == m1, jnp.full((16,), e, jnp.int32), a1)
        masked = [
            jnp.where(jnp.full((16,), e, jnp.int32) == a1, neg_one, vals[e])
            for e in range(N_EXPERTS)
        ]
        m2 = masked[0]
        for v in masked[1:]:
            m2 = jnp.maximum(m2, v)
        a2 = jnp.full((16,), N_EXPERTS - 1, jnp.int32)
        for e in range(N_EXPERTS - 2, -1, -1):
            a2 = jnp.where(masked[e] == m2, jnp.full((16,), e, jnp.int32), a2)
        i1_v[sl] = a1
        i2_v[sl] = a2
        w1_v[sl] = m1
        w2_v[sl] = m2
        return 0

    lax.fori_loop(0, per_w // 16, group, 0)
    pltpu.sync_copy(i1_v, i1_hbm.at[pl.ds(base, per_w)])
    pltpu.sync_copy(i2_v, i2_hbm.at[pl.ds(base, per_w)])
    pltpu.sync_copy(w1_v, w1_hbm.at[pl.ds(base, per_w)])
    pltpu.sync_copy(w2_v, w2_hbm.at[pl.ds(base, per_w)])


@functools.partial(jax.jit, static_argnames=("t", "per_w"))
def _sc_route(scores_flat, t, per_w):
    mesh = plsc.VectorSubcoreMesh(core_axis_name="c", subcore_axis_name="s")
    return pl.kernel(
        functools.partial(_route_body, t, per_w),
        out_type=[
            jax.ShapeDtypeStruct((t,), jnp.int32),
            jax.ShapeDtypeStruct((t,), jnp.int32),
            jax.ShapeDtypeStruct((t,), jnp.float32),
            jax.ShapeDtypeStruct((t,), jnp.float32),
        ],
        mesh=mesh,
        compiler_params=pltpu.CompilerParams(needs_layout_passes=False),
        scratch_types=[
            pltpu.VMEM((N_EXPERTS * per_w,), jnp.float32),
            pltpu.VMEM((per_w,), jnp.int32),
            pltpu.VMEM((per_w,), jnp.int32),
            pltpu.VMEM((per_w,), jnp.float32),
            pltpu.VMEM((per_w,), jnp.float32),
        ],
    )(scores_flat)


def _probe_body(x_ref, o_ref):
    o_ref[...] = x_ref[0, :8, :]


@jax.jit
def _dma_probe(x128):
    bsz, rows, c = x128.shape
    blk6 = 256
    sb = rows // blk6
    return pl.pallas_call(
        _probe_body,
        grid=(bsz, sb),
        in_specs=[pl.BlockSpec((1, blk6, c), lambda i, j: (i, j, 0))],
        out_specs=pl.BlockSpec((8, c), lambda i, j: (i * sb + j, 0)),
        out_shape=jax.ShapeDtypeStruct((bsz * sb * 8, c), jnp.float32),
    )(x128)


def kernel(hidden_states, weight):
    bsz, seq_len, h = hidden_states.shape
    t = bsz * seq_len
    noise_t = _get_noise_t(t)
    probe = _dma_probe(hidden_states.reshape(bsz, seq_len // 4, h * 4))
    scores3 = _tc_scores_t(
        hidden_states, weight, noise_t, per_w=t // 32, blk=1024, q=4)
    scores3 = scores3 + probe[0, 0] * 1e-38
    i1, i2, w1, w2 = _sc_route(scores3.reshape(-1), t=t, per_w=t // 32)
    topk_idx = jnp.stack([i1, i2], axis=1)
    topk_weight = jnp.stack([w1, w2], axis=1)
    return (topk_idx, topk_weight)


# speedup vs baseline: 3.1319x; 3.1319x over previous
import functools

import jax
import jax.numpy as jnp
from jax import lax
from jax.experimental import pallas as pl
from jax.experimental.pallas import tpu as pltpu
from jax.experimental.pallas import tpu_sc as plsc

N_EXPERTS = 8
TOP_K = 2
NOISE_SCALE = 50.0

_noise_cache = {}


def _get_noise_t(tokens: int):
    if tokens not in _noise_cache:
        noise = jax.random.normal(
            jax.random.key(1), (tokens, N_EXPERTS), dtype=jnp.float32
        ) * NOISE_SCALE
        _noise_cache[tokens] = noise.T.copy()
    return _noise_cache[tokens]


def _make_scores_body(q, per_w, step):
    wpb = step // per_w

    def _scores_body(*refs):
        x_refs = refs[:q]
        w_ref, n_ref, o_ref = refs[q], refs[q + 1], refs[q + 2]
        w = w_ref[...]
        parts = [
            lax.dot_general(
                w, x_ref[0],
                (((1,), (1,)), ((), ())),
                preferred_element_type=jnp.float32,
            )
            for x_ref in x_refs
        ]
        logits = (parts[0] if q == 1 else jnp.concatenate(parts, axis=1)) + n_ref[...]
        m = jnp.max(logits, axis=0, keepdims=True)
        e = jnp.exp(logits - m)
        s = jnp.sum(e, axis=0, keepdims=True)
        scores = e / s
        for wk in range(wpb):
            o_ref[wk] = scores[:, wk * per_w:(wk + 1) * per_w]
    return _scores_body


@functools.partial(jax.jit, static_argnames=("blk", "q", "per_w"))
def _tc_scores_t(hidden_states, weight, noise_t, per_w, blk=1024, q=4):
    bsz, seq, h = hidden_states.shape
    step = q * blk
    sb = seq // step
    x_specs = [
        pl.BlockSpec((1, blk, h), lambda i, j, p=p: (i, j * q + p, 0))
        for p in range(q)
    ]
    return pl.pallas_call(
        _make_scores_body(q, per_w, step),
        grid=(bsz, sb),
        in_specs=x_specs + [
            pl.BlockSpec((N_EXPERTS, h), lambda i, j: (0, 0)),
            pl.BlockSpec((N_EXPERTS, step), lambda i, j: (0, i * sb + j)),
        ],
        out_specs=pl.BlockSpec(
            (step // per_w, N_EXPERTS, per_w), lambda i, j: (i * sb + j, 0, 0)),
        out_shape=jax.ShapeDtypeStruct(
            (bsz * seq // per_w, N_EXPERTS, per_w), jnp.float32),
        compiler_params=pltpu.CompilerParams(
            dimension_semantics=("parallel", "parallel")),
    )(*([hidden_states] * q), weight, noise_t)


def _route_body(t, per_w, scores_hbm, i1_hbm, i2_hbm, w1_hbm, w2_hbm,
                sc_v, i1_v, i2_v, w1_v, w2_v):
    wid = lax.axis_index("s") * 2 + lax.axis_index("c")
    base = wid * per_w
    pltpu.sync_copy(
        scores_hbm.at[pl.ds(base * N_EXPERTS, per_w * N_EXPERTS)], sc_v)

    neg_one = jnp.full((16,), -1.0, jnp.float32)

    def group(g, _):
        o = g * 16
        vals = [sc_v[pl.ds(e * per_w + o, 16)] for e in range(N_EXPERTS)]
        sl = pl.ds(o, 16)
        m1 = vals[0]
        for v in vals[1:]:
            m1 = jnp.maximum(m1, v)
        a1 = jnp.full((16,), N_EXPERTS - 1, jnp.int32)
        for e in range(N_EXPERTS - 2, -1, -1):
            a1 = jnp.where(vals[e] == m1, jnp.full((16,), e, jnp.int32), a1)
        masked = [
            jnp.where(jnp.full((16,), e, jnp.int32) == a1, neg_one, vals[e])
            for e in range(N_EXPERTS)
        ]
        m2 = masked[0]
        for v in masked[1:]:
            m2 = jnp.maximum(m2, v)
        a2 = jnp.full((16,), N_EXPERTS - 1, jnp.int32)
        for e in range(N_EXPERTS - 2, -1, -1):
            a2 = jnp.where(masked[e] == m2, jnp.full((16,), e, jnp.int32), a2)
        i1_v[sl] = a1
        i2_v[sl] = a2
        w1_v[sl] = m1
        w2_v[sl] = m2
        return 0

    lax.fori_loop(0, per_w // 16, group, 0)
    pltpu.sync_copy(i1_v, i1_hbm.at[pl.ds(base, per_w)])
    pltpu.sync_copy(i2_v, i2_hbm.at[pl.ds(base, per_w)])
    pltpu.sync_copy(w1_v, w1_hbm.at[pl.ds(base, per_w)])
    pltpu.sync_copy(w2_v, w2_hbm.at[pl.ds(base, per_w)])


@functools.partial(jax.jit, static_argnames=("t", "per_w"))
def _sc_route(scores_flat, t, per_w):
    mesh = plsc.VectorSubcoreMesh(core_axis_name="c", subcore_axis_name="s")
    return pl.kernel(
        functools.partial(_route_body, t, per_w),
        out_type=[
            jax.ShapeDtypeStruct((t,), jnp.int32),
            jax.ShapeDtypeStruct((t,), jnp.int32),
            jax.ShapeDtypeStruct((t,), jnp.float32),
            jax.ShapeDtypeStruct((t,), jnp.float32),
        ],
        mesh=mesh,
        compiler_params=pltpu.CompilerParams(needs_layout_passes=False),
        scratch_types=[
            pltpu.VMEM((N_EXPERTS * per_w,), jnp.float32),
            pltpu.VMEM((per_w,), jnp.int32),
            pltpu.VMEM((per_w,), jnp.int32),
            pltpu.VMEM((per_w,), jnp.float32),
            pltpu.VMEM((per_w,), jnp.float32),
        ],
    )(scores_flat)


def kernel(hidden_states, weight):
    bsz, seq_len, h = hidden_states.shape
    t = bsz * seq_len
    noise_t = _get_noise_t(t)
    scores3 = _tc_scores_t(
        hidden_states, weight, noise_t, per_w=t // 32, blk=512, q=8)
    i1, i2, w1, w2 = _sc_route(scores3.reshape(-1), t=t, per_w=t // 32)
    topk_idx = jnp.stack([i1, i2], axis=1)
    topk_weight = jnp.stack([w1, w2], axis=1)
    return (topk_idx, topk_weight)
